# baseline (speedup 1.0000x reference)
import functools
import math

import jax
import jax.numpy as jnp
from jax import lax
from jax.experimental import pallas as pl
from jax.experimental.pallas import tpu as pltpu

COS_T = math.cos(0.2)
Q = 4096
T = 128
NC = 10000
NCP = 10240
NHALF = NCP // 2
UNROLL = 32


def _prep_kernel(cdx_ref, cdy_ref, typ_ref, ccos_ref, csin_ref):
    cdx = cdx_ref[...]
    cdy = cdy_ref[...]
    typ = typ_ref[...]
    valid = (typ == 1) | (typ == 2)
    n2 = cdx * cdx + cdy * cdy
    nz = n2 > 0.0
    r = lax.rsqrt(n2)
    ccos = jnp.where(valid & nz, cdx * r, jnp.where(valid, 1.0, 0.0))
    csin = jnp.where(valid & nz, cdy * r, 0.0)
    ccos_ref[...] = ccos.astype(jnp.float32)
    csin_ref[...] = csin.astype(jnp.float32)


def _main_kernel(qx_ref, qy_ref, cx_ref, cy_ref, ccos_ref, csin_ref,
                 out_ref):
    qx = qx_ref[...]
    qy = qy_ref[...]
    dqx = pltpu.roll(qx, T - 1, 1) - qx
    dqy = pltpu.roll(qy, T - 1, 1) - qy
    lane = lax.broadcasted_iota(jnp.int32, (32, T), 1)
    is_last = lane == (T - 1)
    dqx = jnp.where(is_last, pltpu.roll(dqx, 1, 1), dqx)
    dqy = jnp.where(is_last, pltpu.roll(dqy, 1, 1), dqy)
    n2 = dqx * dqx + dqy * dqy
    nz = n2 > 0.0
    r = lax.rsqrt(n2)
    qcos = jnp.where(nz, dqx * r, 1.0)
    qsin = jnp.where(nz, dqy * r, 0.0)

    half = pl.program_id(0) * NHALF

    def body(i, acc):
        base = half + i * UNROLL
        mds = []
        for u in range(UNROLL):
            k = base + u
            dx = qx - cx_ref[0, k]
            dy = qy - cy_ref[0, k]
            d2 = dx * dx + dy * dy
            dot = qcos * ccos_ref[0, k] + qsin * csin_ref[0, k]
            mds.append(jnp.where(dot > COS_T, d2, jnp.inf))
        while len(mds) > 1:
            nxt = [jnp.minimum(mds[j], mds[j + 1])
                   for j in range(0, len(mds) - 1, 2)]
            if len(mds) % 2:
                nxt.append(mds[-1])
            mds = nxt
        return jnp.minimum(acc, mds[0])

    init = jnp.full((32, T), jnp.inf, jnp.float32)
    acc = lax.fori_loop(0, NHALF // UNROLL, body, init)
    out_ref[...] = acc.reshape(1, 32, T)


def _merge_kernel(acc_ref, out_ref):
    m2 = jnp.minimum(acc_ref[0], acc_ref[1])
    dist = jnp.where(m2 < 25.0, jnp.sqrt(m2), jnp.inf)
    out_ref[...] = jnp.sum(dist).reshape(1, 1)


@jax.jit
def kernel(xy, types, xyz, dir):
    xy = xy.astype(jnp.float32)
    xyz = xyz.astype(jnp.float32)
    dir = dir.astype(jnp.float32)
    typ = types.astype(jnp.int32)

    pad = NCP - NC
    cdx = jnp.pad(dir[:, 0], (0, pad)).reshape(80, 128)
    cdy = jnp.pad(dir[:, 1], (0, pad)).reshape(80, 128)
    typ2 = jnp.pad(typ, (0, pad)).reshape(80, 128)

    ccos, csin = pl.pallas_call(
        _prep_kernel,
        out_shape=[jax.ShapeDtypeStruct((80, 128), jnp.float32)] * 2,
    )(cdx, cdy, typ2)

    qx = xy[:, :, 0]
    qy = xy[:, :, 1]
    cx = jnp.pad(xyz[:, 0], (0, pad)).reshape(1, NCP)
    cy = jnp.pad(xyz[:, 1], (0, pad)).reshape(1, NCP)
    ccos = ccos.reshape(1, NCP)
    csin = csin.reshape(1, NCP)

    q_spec = pl.BlockSpec((32, T), lambda i: (0, 0))
    c_spec = pl.BlockSpec(memory_space=pltpu.SMEM)
    accs = pl.pallas_call(
        _main_kernel,
        grid=(2,),
        in_specs=[q_spec, q_spec, c_spec, c_spec, c_spec, c_spec],
        out_specs=pl.BlockSpec((1, 32, T), lambda i: (i, 0, 0)),
        out_shape=jax.ShapeDtypeStruct((2, 32, T), jnp.float32),
        compiler_params=pltpu.CompilerParams(
            dimension_semantics=("parallel",),
        ),
    )(qx, qy, cx, cy, ccos, csin)

    total = pl.pallas_call(
        _merge_kernel,
        out_shape=jax.ShapeDtypeStruct((1, 1), jnp.float32),
    )(accs)

    return total[0, 0] / Q


# speedup vs baseline: 1.0360x; 1.0360x over previous
import functools
import math

import jax
import jax.numpy as jnp
from jax import lax
from jax.experimental import pallas as pl
from jax.experimental.pallas import tpu as pltpu

COS_T = math.cos(0.2)
Q = 4096
T = 128
NC = 10000
NCP = 10240
NHALF = NCP // 2
UNROLL = 16


def _prep_kernel(cdx_ref, cdy_ref, typ_ref, ccos_ref, csin_ref):
    cdx = cdx_ref[...]
    cdy = cdy_ref[...]
    typ = typ_ref[...]
    valid = (typ == 1) | (typ == 2)
    n2 = cdx * cdx + cdy * cdy
    nz = n2 > 0.0
    r = lax.rsqrt(n2)
    ccos = jnp.where(valid & nz, cdx * r, jnp.where(valid, 1.0, 0.0))
    csin = jnp.where(valid & nz, cdy * r, 0.0)
    ccos_ref[...] = ccos.astype(jnp.float32)
    csin_ref[...] = csin.astype(jnp.float32)


def _main_kernel(qx_ref, qy_ref, cx_ref, cy_ref, ccos_ref, csin_ref,
                 out_ref):
    qx = qx_ref[...]
    qy = qy_ref[...]
    dqx = pltpu.roll(qx, T - 1, 1) - qx
    dqy = pltpu.roll(qy, T - 1, 1) - qy
    lane = lax.broadcasted_iota(jnp.int32, (32, T), 1)
    is_last = lane == (T - 1)
    dqx = jnp.where(is_last, pltpu.roll(dqx, 1, 1), dqx)
    dqy = jnp.where(is_last, pltpu.roll(dqy, 1, 1), dqy)
    n2 = dqx * dqx + dqy * dqy
    nz = n2 > 0.0
    r = lax.rsqrt(n2)
    qcos = jnp.where(nz, dqx * r, 1.0)
    qsin = jnp.where(nz, dqy * r, 0.0)

    half = pl.program_id(0) * NHALF

    def body(i, acc):
        base = half + i * UNROLL
        mds = []
        for u in range(UNROLL):
            k = base + u
            dx = qx - cx_ref[0, k]
            dy = qy - cy_ref[0, k]
            d2 = dx * dx + dy * dy
            dot = qcos * ccos_ref[0, k] + qsin * csin_ref[0, k]
            mds.append(jnp.where(dot > COS_T, d2, jnp.inf))
        while len(mds) > 1:
            nxt = [jnp.minimum(mds[j], mds[j + 1])
                   for j in range(0, len(mds) - 1, 2)]
            if len(mds) % 2:
                nxt.append(mds[-1])
            mds = nxt
        return jnp.minimum(acc, mds[0])

    init = jnp.full((32, T), jnp.inf, jnp.float32)
    acc = lax.fori_loop(0, NHALF // UNROLL, body, init)
    out_ref[...] = acc.reshape(1, 32, T)


def _merge_kernel(acc_ref, out_ref):
    m2 = jnp.minimum(acc_ref[0], acc_ref[1])
    dist = jnp.where(m2 < 25.0, jnp.sqrt(m2), jnp.inf)
    out_ref[...] = jnp.sum(dist).reshape(1, 1)


@jax.jit
def kernel(xy, types, xyz, dir):
    xy = xy.astype(jnp.float32)
    xyz = xyz.astype(jnp.float32)
    dir = dir.astype(jnp.float32)
    typ = types.astype(jnp.int32)

    pad = NCP - NC
    cdx = jnp.pad(dir[:, 0], (0, pad)).reshape(80, 128)
    cdy = jnp.pad(dir[:, 1], (0, pad)).reshape(80, 128)
    typ2 = jnp.pad(typ, (0, pad)).reshape(80, 128)

    ccos, csin = pl.pallas_call(
        _prep_kernel,
        out_shape=[jax.ShapeDtypeStruct((80, 128), jnp.float32)] * 2,
    )(cdx, cdy, typ2)

    qx = xy[:, :, 0]
    qy = xy[:, :, 1]
    cx = jnp.pad(xyz[:, 0], (0, pad)).reshape(1, NCP)
    cy = jnp.pad(xyz[:, 1], (0, pad)).reshape(1, NCP)
    ccos = ccos.reshape(1, NCP)
    csin = csin.reshape(1, NCP)

    q_spec = pl.BlockSpec((32, T), lambda i: (0, 0))
    c_spec = pl.BlockSpec(memory_space=pltpu.SMEM)
    accs = pl.pallas_call(
        _main_kernel,
        grid=(2,),
        in_specs=[q_spec, q_spec, c_spec, c_spec, c_spec, c_spec],
        out_specs=pl.BlockSpec((1, 32, T), lambda i: (i, 0, 0)),
        out_shape=jax.ShapeDtypeStruct((2, 32, T), jnp.float32),
        compiler_params=pltpu.CompilerParams(
            dimension_semantics=("parallel",),
        ),
    )(qx, qy, cx, cy, ccos, csin)

    total = pl.pallas_call(
        _merge_kernel,
        out_shape=jax.ShapeDtypeStruct((1, 1), jnp.float32),
    )(accs)

    return total[0, 0] / Q
